# baseline (speedup 1.0000x reference)
import numpy as np
import jax
import jax.numpy as jnp
from jax.experimental import pallas as pl
from jax.experimental.pallas import tpu as pltpu

_K3A = (4095997477, 317277840)
_K4A = (1820970612, 3729538270)
_K3B = (257214496, 567757975)
_K4B = (1383271662, 289976367)
_SF_A = np.float32(0.2342308759689331)
_NF_A = np.float32(0.4123215079307556)
_SF_B = np.float32(0.22688937187194824)
_NF_B = np.float32(0.1473727971315384)

_B = 16
_N = 100000
_DEPTH = 3
_MISSING = np.float32(-1.0)
_W = 1024


def _threefry_xor(key, cnt):
    k1, k2 = key
    ks = (np.uint32(k1), np.uint32(k2),
          np.uint32((k1 ^ k2 ^ 0x1BD11BDA) & 0xFFFFFFFF))
    rot = ((13, 15, 26, 6), (17, 29, 16, 24))
    x0 = jnp.full_like(cnt, ks[0])
    x1 = cnt + ks[1]
    for i in range(5):
        for r in rot[i % 2]:
            x0 = x0 + x1
            x1 = (x1 << np.uint32(r)) | (x1 >> np.uint32(32 - r))
            x1 = x1 ^ x0
        x0 = x0 + ks[(i + 1) % 3]
        x1 = x1 + np.uint32((int(ks[(i + 2) % 3]) + i + 1) & 0xFFFFFFFF)
    return x0 ^ x1


def _uniform01(key, cnt):
    bits = _threefry_xor(key, cnt)
    fb = (bits >> np.uint32(9)) | np.uint32(0x3F800000)
    return pltpu.bitcast(fb, jnp.float32) - np.float32(1.0)


def _sparse_pass(x0, x1, cnt, k3, k4, sf, nf):
    one = np.float32(1.0)
    half = np.float32(0.5)
    two = np.float32(2.0)
    u_b = _uniform01(k3, cnt)
    bn = _uniform01(k4, cnt)
    mask = jnp.where(u_b < sf, np.float32(0.0), one)
    nm = jnp.where(bn < nf, one, np.float32(0.0))
    d = x0 - half
    a = jnp.abs(d)
    s = jnp.floor(nm * bn / nf - half) + half
    inter = d * a * two + np.float32(-2.0) * (a - half) * s * two / two
    noised = x0 - inter * nm
    miss = x0 == _MISSING
    noised = jnp.where(miss, _MISSING, noised)
    spars = noised * mask * two - _MISSING * (mask - one)
    spars = jnp.where(miss, _MISSING, spars)
    cls = spars.astype(jnp.int32)
    return cls, one - mask


def _interleave5(ch):
    stacked = jnp.stack(ch, axis=-1)
    return stacked.reshape(_B, -1)


def _body(x0_ref, x1_ref, o_ref):
    j = pl.program_id(0)
    n0 = (j * _W).astype(jnp.uint32)
    x0 = x0_ref[...]
    x1 = x1_ref[...]
    row = jax.lax.broadcasted_iota(jnp.uint32, (_B, _W), 0)
    col = jax.lax.broadcasted_iota(jnp.uint32, (_B, _W), 1)
    cnt = row * np.uint32(_N) + col + n0

    for half_idx, (k3, k4, sf, nf) in enumerate(
            ((_K3A, _K4A, _SF_A, _NF_A), (_K3B, _K4B, _SF_B, _NF_B))):
        cls, inv_mask = _sparse_pass(x0, x1, cnt, k3, k4, sf, nf)
        clsf = [(cls == c).astype(jnp.float32) for c in range(_DEPTH)]
        tile = _interleave5(clsf + [inv_mask, x1])
        o_ref[pl.ds(half_idx * _B, _B), :] = tile


def kernel(inputs):
    x0 = inputs[:, :, 0]
    x1 = inputs[:, :, 1]
    grid = (_N + _W - 1) // _W
    out2d = pl.pallas_call(
        _body,
        grid=(grid,),
        in_specs=[
            pl.BlockSpec((_B, _W), lambda j: (0, j)),
            pl.BlockSpec((_B, _W), lambda j: (0, j)),
        ],
        out_specs=pl.BlockSpec((2 * _B, 5 * _W), lambda j: (0, j)),
        out_shape=jax.ShapeDtypeStruct((2 * _B, 5 * _N), jnp.float32),
    )(x0, x1)
    return out2d.reshape(2 * _B, _N, 5)


# speedup vs baseline: 22.4745x; 22.4745x over previous
import numpy as np
import jax
import jax.numpy as jnp
from jax.experimental import pallas as pl
from jax.experimental.pallas import tpu as pltpu

_K3A = (4095997477, 317277840)
_K4A = (1820970612, 3729538270)
_K3B = (257214496, 567757975)
_K4B = (1383271662, 289976367)
_SF_A = np.float32(0.2342308759689331)
_NF_A = np.float32(0.4123215079307556)
_SF_B = np.float32(0.22688937187194824)
_NF_B = np.float32(0.1473727971315384)

_B = 16
_N = 100000
_DEPTH = 3
_MISSING = np.float32(-1.0)
_W = 2048


def _threefry_xor(key, cnt):
    k1, k2 = key
    ks = (np.uint32(k1), np.uint32(k2),
          np.uint32((k1 ^ k2 ^ 0x1BD11BDA) & 0xFFFFFFFF))
    rot = ((13, 15, 26, 6), (17, 29, 16, 24))
    x0 = jnp.full_like(cnt, ks[0])
    x1 = cnt + ks[1]
    for i in range(5):
        for r in rot[i % 2]:
            x0 = x0 + x1
            x1 = (x1 << np.uint32(r)) | (x1 >> np.uint32(32 - r))
            x1 = x1 ^ x0
        x0 = x0 + ks[(i + 1) % 3]
        x1 = x1 + np.uint32((int(ks[(i + 2) % 3]) + i + 1) & 0xFFFFFFFF)
    return x0 ^ x1


def _uniform01(key, cnt):
    bits = _threefry_xor(key, cnt)
    fb = (bits >> np.uint32(9)) | np.uint32(0x3F800000)
    return pltpu.bitcast(fb, jnp.float32) - np.float32(1.0)


def _sparse_pass(x0, cnt, k3, k4, sf, nf):
    one = np.float32(1.0)
    half = np.float32(0.5)
    two = np.float32(2.0)
    u_b = _uniform01(k3, cnt)
    bn = _uniform01(k4, cnt)
    mask = jnp.where(u_b < sf, np.float32(0.0), one)
    nm = jnp.where(bn < nf, one, np.float32(0.0))
    d = x0 - half
    a = jnp.abs(d)
    s = jnp.floor(nm * bn / nf - half) + half
    inter = d * a * two + np.float32(-2.0) * (a - half) * s * two / two
    noised = x0 - inter * nm
    miss = x0 == _MISSING
    noised = jnp.where(miss, _MISSING, noised)
    spars = noised * mask * two - _MISSING * (mask - one)
    spars = jnp.where(miss, _MISSING, spars)
    cls = spars.astype(jnp.int32)
    return cls, one - mask


def _body(x0_ref, x1_ref, o_ref):
    j = pl.program_id(0)
    n0 = (j * _W).astype(jnp.uint32)
    x0 = x0_ref[...]
    x1 = x1_ref[...]
    row = jax.lax.broadcasted_iota(jnp.uint32, (_B, _W), 0)
    col = jax.lax.broadcasted_iota(jnp.uint32, (_B, _W), 1)
    cnt = row * np.uint32(_N) + col + n0

    for half_idx, (k3, k4, sf, nf) in enumerate(
            ((_K3A, _K4A, _SF_A, _NF_A), (_K3B, _K4B, _SF_B, _NF_B))):
        cls, inv_mask = _sparse_pass(x0, cnt, k3, k4, sf, nf)
        rows = pl.ds(half_idx * _B, _B)
        for c in range(_DEPTH):
            o_ref[c, rows, :] = (cls == c).astype(jnp.float32)
        o_ref[_DEPTH, rows, :] = inv_mask
        o_ref[_DEPTH + 1, rows, :] = x1


def kernel(inputs):
    x0 = inputs[:, :, 0]
    x1 = inputs[:, :, 1]
    grid = (_N + _W - 1) // _W
    out5 = pl.pallas_call(
        _body,
        grid=(grid,),
        in_specs=[
            pl.BlockSpec((_B, _W), lambda j: (0, j)),
            pl.BlockSpec((_B, _W), lambda j: (0, j)),
        ],
        out_specs=pl.BlockSpec((5, 2 * _B, _W), lambda j: (0, 0, j)),
        out_shape=jax.ShapeDtypeStruct((5, 2 * _B, _N), jnp.float32),
    )(x0, x1)
    return out5.transpose(1, 2, 0)
